# Initial kernel scaffold; baseline (speedup 1.0000x reference)
import jax
import jax.numpy as jnp
from jax.experimental import pallas as pl


def kernel(x, edge_index, W0, b0, W4, b4):
    raise NotImplementedError("write your pallas kernel here")


# speedup vs baseline: 15.4521x; 15.4521x over previous
import functools

import jax
import jax.numpy as jnp
from jax import lax
from jax.experimental import pallas as pl
from jax.experimental.pallas import tpu as pltpu
from jax.experimental.pallas import tpu_sc as plsc

NC = 2
NS = 16
NWK = NC * NS
WIN = 128
GR = 240


def _mesh():
    return plsc.VectorSubcoreMesh(
        core_axis_name="c", subcore_axis_name="s",
        num_cores=NC, num_subcores=NS)


def _make_deg(n, nw):
    acc_rows = n + GR
    zrows = acc_rows // NS

    @functools.partial(
        pl.kernel, mesh=_mesh(),
        out_type=[jax.ShapeDtypeStruct((acc_rows, 16), jnp.float32),
                  jax.ShapeDtypeStruct((acc_rows, 16), jnp.float32)],
        scratch_types=[
            pltpu.VMEM((WIN, 16), jnp.float32),
            pltpu.VMEM((WIN, 16), jnp.float32),
            pltpu.VMEM((WIN,), jnp.int32),
            pltpu.VMEM((WIN,), jnp.int32),
            pltpu.VMEM_SHARED((acc_rows, 16), jnp.float32),
            pltpu.SemaphoreType.DMA,
            pltpu.SemaphoreType.DMA,
        ],
    )
    def deg_kernel(dst_hbm, ones_hbm, z_hbm, out0, out1,
                   onesb, zb, db0, db1, acc, s0, s1):
        c = lax.axis_index("c")
        s = lax.axis_index("s")
        wid = s * NC + c
        pltpu.sync_copy(ones_hbm, onesb)
        pltpu.sync_copy(z_hbm, zb)
        for k in range(zrows // WIN):
            pltpu.sync_copy(zb, acc.at[pl.ds(s * zrows + k * WIN, WIN)])
        plsc.subcore_barrier()

        base = wid * nw
        dbufs = (db0, db1)
        sems = (s0, s1)

        def body(j, carry):
            for b in range(2):
                i = 2 * j + b

                @pl.when(i >= 2)
                def _():
                    pltpu.make_async_copy(
                        onesb, acc.at[dbufs[b]], sems[b]).wait()

                off = pl.multiple_of((base + i) * WIN, WIN)
                pltpu.sync_copy(dst_hbm.at[pl.ds(off, WIN)], dbufs[b])
                pltpu.async_copy(onesb, acc.at[dbufs[b]], sems[b], add=True)
            return carry

        lax.fori_loop(0, nw // 2, body, 0)
        pltpu.make_async_copy(onesb, acc.at[db0], s0).wait()
        pltpu.make_async_copy(onesb, acc.at[db1], s1).wait()
        plsc.subcore_barrier()

        for k in range(zrows // WIN):
            r0 = s * zrows + k * WIN
            pltpu.sync_copy(acc.at[pl.ds(r0, WIN)], zb)

            @pl.when(c == 0)
            def _():
                pltpu.sync_copy(zb, out0.at[pl.ds(r0, WIN)])

            @pl.when(c == 1)
            def _():
                pltpu.sync_copy(zb, out1.at[pl.ds(r0, WIN)])

    return deg_kernel


def _make_prop(n, cdim, nw):
    acc_rows = n + GR
    zrows = acc_rows // NS

    @functools.partial(
        pl.kernel, mesh=_mesh(),
        out_type=[jax.ShapeDtypeStruct((acc_rows, cdim), jnp.float32),
                  jax.ShapeDtypeStruct((acc_rows, cdim), jnp.float32)],
        scratch_types=[
            pltpu.VMEM((WIN,), jnp.int32),
            pltpu.VMEM((WIN,), jnp.int32),
            pltpu.VMEM((WIN,), jnp.int32),
            pltpu.VMEM((WIN,), jnp.int32),
            pltpu.VMEM((WIN, cdim), jnp.float32),
            pltpu.VMEM((WIN, cdim), jnp.float32),
            pltpu.VMEM_SHARED((acc_rows, cdim), jnp.float32),
            pltpu.SemaphoreType.DMA,
            pltpu.SemaphoreType.DMA,
            pltpu.SemaphoreType.DMA,
            pltpu.SemaphoreType.DMA,
        ],
    )
    def prop_kernel(src_hbm, dst_hbm, g_hbm, z_hbm, out0, out1,
                    sb0, sb1, db0, db1, rows0, rows1, acc,
                    gs0, gs1, ss0, ss1):
        c = lax.axis_index("c")
        s = lax.axis_index("s")
        wid = s * NC + c
        pltpu.sync_copy(z_hbm, rows0)
        for k in range(zrows // WIN):
            pltpu.sync_copy(rows0, acc.at[pl.ds(s * zrows + k * WIN, WIN)])
        plsc.subcore_barrier()

        base = wid * nw
        sbufs = (sb0, sb1)
        dbufs = (db0, db1)
        rows = (rows0, rows1)
        gsem = (gs0, gs1)
        ssem = (ss0, ss1)

        pltpu.sync_copy(src_hbm.at[pl.ds(pl.multiple_of(base * WIN, WIN), WIN)],
                        sb0)
        pltpu.async_copy(g_hbm.at[sb0], rows0, gs0)

        def body(j, carry):
            for b in range(2):
                i = 2 * j + b
                nb = 1 - b

                @pl.when(i >= 1)
                def _():
                    pltpu.make_async_copy(
                        rows[nb], acc.at[dbufs[nb]], ssem[nb]).wait()

                @pl.when(i + 1 < nw)
                def _():
                    off = pl.multiple_of((base + i + 1) * WIN, WIN)
                    pltpu.sync_copy(src_hbm.at[pl.ds(off, WIN)], sbufs[nb])
                    pltpu.async_copy(g_hbm.at[sbufs[nb]], rows[nb], gsem[nb])

                pltpu.make_async_copy(g_hbm.at[sbufs[b]], rows[b],
                                      gsem[b]).wait()
                off = pl.multiple_of((base + i) * WIN, WIN)
                pltpu.sync_copy(dst_hbm.at[pl.ds(off, WIN)], dbufs[b])
                pltpu.async_copy(rows[b], acc.at[dbufs[b]], ssem[b], add=True)
            return carry

        lax.fori_loop(0, nw // 2, body, 0)
        pltpu.make_async_copy(rows1, acc.at[db1], ssem[1]).wait()
        plsc.subcore_barrier()

        for k in range(zrows // WIN):
            r0 = s * zrows + k * WIN
            pltpu.sync_copy(acc.at[pl.ds(r0, WIN)], rows0)

            @pl.when(c == 0)
            def _():
                pltpu.sync_copy(rows0, out0.at[pl.ds(r0, WIN)])

            @pl.when(c == 1)
            def _():
                pltpu.sync_copy(rows0, out1.at[pl.ds(r0, WIN)])

    return prop_kernel


_BM = 1000


def _tc_call(body, n, out_shapes, in_specs, out_specs):
    return pl.pallas_call(
        body,
        grid=(n // _BM,),
        in_specs=in_specs,
        out_specs=out_specs,
        out_shape=out_shapes,
    )


def _rowspec(c):
    return pl.BlockSpec((_BM, c), lambda i: (i, 0))


def _fullspec(r, c):
    return pl.BlockSpec((r, c), lambda i: (0, 0))


def _dis_body(d0, d1, x, dis_o, g_o):
    deg = d0[:, 0:1] + d1[:, 0:1]
    dis = lax.rsqrt(deg)
    dis_o[...] = dis
    g_o[...] = dis * x[...]


def _mix_chain_body(a0, a1, dis, g_o):
    d = dis[...]
    g_o[...] = d * d * (a0[...] + a1[...])


def _mix_mid_body(a0, a1, dis, h, g_o):
    d = dis[...]
    g_o[...] = d * (0.8 * (d * (a0[...] + a1[...])) + 0.2 * h[...])


def _mix_end_body(a0, a1, dis, h, h_o, g_o):
    d = dis[...]
    z = 0.8 * (d * (a0[...] + a1[...])) + 0.2 * h[...]
    z = jnp.where(z >= 0, z, 0.01 * z)
    h_o[...] = z
    g_o[...] = d * z


def _mm0_body(a0, a1, dis, w, b, h_o, g_o):
    d = dis[...]
    sarr = d * (a0[...] + a1[...])
    h = jnp.dot(sarr, w[...], preferred_element_type=jnp.float32) + b[...]
    h_o[...] = h
    g_o[...] = d * h


def _mmf_body(a0, a1, dis, w, b, o_ref):
    d = dis[...]
    sarr = d * (a0[...] + a1[...])
    h = jnp.dot(sarr, w[...], preferred_element_type=jnp.float32) + b[...]
    m = jnp.max(h, axis=1, keepdims=True)
    z = h - m
    o_ref[...] = z - jnp.log(jnp.sum(jnp.exp(z), axis=1, keepdims=True))


def kernel(x, edge_index, W0, b0, W4, b4):
    n, cdim = x.shape
    e = edge_index.shape[1]
    outc = W4.shape[1]
    f32 = jnp.float32

    ea = e + n
    nw = -(-ea // (NWK * WIN))
    if nw % 2:
        nw += 1
    ea_pad = NWK * nw * WIN
    npad = ea_pad - ea

    ar = jnp.arange(n, dtype=jnp.int32)
    padi = jnp.arange(npad, dtype=jnp.int32)
    src = jnp.concatenate([edge_index[0], ar, (padi * 7919) % n])
    dst = jnp.concatenate([edge_index[1], ar, n + (padi % GR)])
    z128 = jnp.zeros((WIN, cdim), f32)
    ones16 = jnp.ones((WIN, 16), f32)
    z16 = jnp.zeros((WIN, 16), f32)

    deg0, deg1 = _make_deg(n, nw)(dst, ones16, z16)

    dis, g = _tc_call(
        _dis_body, n,
        [jax.ShapeDtypeStruct((n, 1), f32),
         jax.ShapeDtypeStruct((n, cdim), f32)],
        [_rowspec(16), _rowspec(16), _rowspec(cdim)],
        [_rowspec(1), _rowspec(cdim)],
    )(deg0, deg1, x)

    prop = _make_prop(n, cdim, nw)
    gshape = jax.ShapeDtypeStruct((n, cdim), f32)

    mix_chain = _tc_call(
        _mix_chain_body, n, gshape,
        [_rowspec(cdim), _rowspec(cdim), _rowspec(1)], _rowspec(cdim))
    mix_mid = _tc_call(
        _mix_mid_body, n, gshape,
        [_rowspec(cdim), _rowspec(cdim), _rowspec(1), _rowspec(cdim)],
        _rowspec(cdim))
    mix_end = _tc_call(
        _mix_end_body, n, [gshape, gshape],
        [_rowspec(cdim), _rowspec(cdim), _rowspec(1), _rowspec(cdim)],
        [_rowspec(cdim), _rowspec(cdim)])

    a0, a1 = prop(src, dst, g, z128)
    g = mix_chain(a0, a1, dis)
    a0, a1 = prop(src, dst, g, z128)
    h, g = _tc_call(
        _mm0_body, n, [gshape, gshape],
        [_rowspec(cdim), _rowspec(cdim), _rowspec(1),
         _fullspec(cdim, cdim), _fullspec(1, cdim)],
        [_rowspec(cdim), _rowspec(cdim)],
    )(a0, a1, dis, W0, b0.reshape(1, cdim))

    for _ in range(3):
        a0, a1 = prop(src, dst, g, z128)
        g = mix_mid(a0, a1, dis, h)
        a0, a1 = prop(src, dst, g, z128)
        h, g = mix_end(a0, a1, dis, h)

    a0, a1 = prop(src, dst, g, z128)
    g = mix_chain(a0, a1, dis)
    a0, a1 = prop(src, dst, g, z128)
    out = _tc_call(
        _mmf_body, n, jax.ShapeDtypeStruct((n, outc), f32),
        [_rowspec(cdim), _rowspec(cdim), _rowspec(1),
         _fullspec(cdim, outc), _fullspec(1, outc)],
        _rowspec(outc),
    )(a0, a1, dis, W4, b4.reshape(1, outc))
    return out


# speedup vs baseline: 19.9091x; 1.2884x over previous
import functools

import jax
import jax.numpy as jnp
from jax import lax
from jax.experimental import pallas as pl
from jax.experimental.pallas import tpu as pltpu
from jax.experimental.pallas import tpu_sc as plsc

NC = 2
NS = 16
NWK = NC * NS
WIN = 128
GR = 112


def _chunks(rows):
    out, off = [], 0
    while off < rows:
        sz = min(WIN, rows - off)
        out.append((off, sz))
        off += sz
    return out


def _mesh():
    return plsc.VectorSubcoreMesh(
        core_axis_name="c", subcore_axis_name="s",
        num_cores=NC, num_subcores=NS)


def _fill_win(dst_ref, src_ref, i):
    for q in range(WIN // 16):
        dst_ref[pl.ds(q * 16, 16)] = src_ref[pl.ds(i * WIN + q * 16, 16)]


def _make_deg(n, nw):
    acc_rows = n + GR
    zrows = acc_rows // NS
    chunk = nw * WIN

    @functools.partial(
        pl.kernel, mesh=_mesh(),
        out_type=[jax.ShapeDtypeStruct((acc_rows, 16), jnp.float32),
                  jax.ShapeDtypeStruct((acc_rows, 16), jnp.float32)],
        scratch_types=[
            pltpu.VMEM((WIN, 16), jnp.float32),
            pltpu.VMEM((WIN, 16), jnp.float32),
            pltpu.VMEM((chunk,), jnp.int32),
            pltpu.VMEM((WIN,), jnp.int32),
            pltpu.VMEM((WIN,), jnp.int32),
            pltpu.VMEM_SHARED((acc_rows, 16), jnp.float32),
            pltpu.SemaphoreType.DMA,
            pltpu.SemaphoreType.DMA,
        ],
    )
    def deg_kernel(dst_hbm, ones_hbm, z_hbm, out0, out1,
                   onesb, zb, dall, db0, db1, acc, s0, s1):
        c = lax.axis_index("c")
        s = lax.axis_index("s")
        wid = s * NC + c
        pltpu.sync_copy(ones_hbm, onesb)
        pltpu.sync_copy(z_hbm, zb)
        pltpu.sync_copy(
            dst_hbm.at[pl.ds(pl.multiple_of(wid * chunk, WIN), chunk)], dall)
        for off, sz in _chunks(zrows):
            pltpu.sync_copy(zb.at[pl.ds(0, sz)],
                            acc.at[pl.ds(s * zrows + off, sz)])
        plsc.subcore_barrier()

        dbufs = (db0, db1)
        sems = (s0, s1)

        def body(j, carry):
            for b in range(2):
                i = 2 * j + b

                @pl.when(i >= 2)
                def _():
                    pltpu.make_async_copy(
                        onesb, acc.at[dbufs[b]], sems[b]).wait()

                _fill_win(dbufs[b], dall, i)
                pltpu.async_copy(onesb, acc.at[dbufs[b]], sems[b], add=True)
            return carry

        lax.fori_loop(0, nw // 2, body, 0)
        pltpu.make_async_copy(onesb, acc.at[db0], s0).wait()
        pltpu.make_async_copy(onesb, acc.at[db1], s1).wait()
        plsc.subcore_barrier()

        for off, sz in _chunks(zrows):
            r0 = s * zrows + off
            pltpu.sync_copy(acc.at[pl.ds(r0, sz)], zb.at[pl.ds(0, sz)])

            @pl.when(c == 0)
            def _():
                pltpu.sync_copy(zb.at[pl.ds(0, sz)], out0.at[pl.ds(r0, sz)])

            @pl.when(c == 1)
            def _():
                pltpu.sync_copy(zb.at[pl.ds(0, sz)], out1.at[pl.ds(r0, sz)])

    return deg_kernel


def _make_prop(n, cdim, nw):
    acc_rows = n + GR
    zrows = acc_rows // NS

    chunk = nw * WIN
    nwh = nw // 2
    chunk2 = nwh * WIN

    @functools.partial(
        pl.kernel, mesh=_mesh(),
        out_type=[jax.ShapeDtypeStruct((acc_rows, cdim), jnp.float32),
                  jax.ShapeDtypeStruct((acc_rows, cdim), jnp.float32)],
        scratch_types=[
            pltpu.VMEM((chunk2,), jnp.int32),
            pltpu.VMEM((chunk2,), jnp.int32),
            pltpu.VMEM((WIN,), jnp.int32),
            pltpu.VMEM((WIN,), jnp.int32),
            pltpu.VMEM((WIN,), jnp.int32),
            pltpu.VMEM((WIN,), jnp.int32),
            pltpu.VMEM((WIN, cdim), jnp.float32),
            pltpu.VMEM((WIN, cdim), jnp.float32),
            pltpu.VMEM_SHARED((acc_rows, cdim), jnp.float32),
            pltpu.SemaphoreType.DMA,
            pltpu.SemaphoreType.DMA,
            pltpu.SemaphoreType.DMA,
            pltpu.SemaphoreType.DMA,
        ],
    )
    def prop_kernel(src_hbm, dst_hbm, g_hbm, z_hbm, out0, out1,
                    sall, dall, sb0, sb1, db0, db1, rows0, rows1, acc,
                    gs0, gs1, ss0, ss1):
        c = lax.axis_index("c")
        s = lax.axis_index("s")
        wid = s * NC + c
        pltpu.sync_copy(z_hbm, rows0)
        for k in range(zrows // WIN):
            pltpu.sync_copy(rows0, acc.at[pl.ds(s * zrows + k * WIN, WIN)])
        plsc.subcore_barrier()

        sbufs = (sb0, sb1)
        dbufs = (db0, db1)
        rows = (rows0, rows1)
        gsem = (gs0, gs1)
        ssem = (ss0, ss1)

        def body(j, carry):
            for b in range(2):
                i = 2 * j + b
                nb = 1 - b

                @pl.when(i >= 1)
                def _():
                    pltpu.make_async_copy(
                        rows[nb], acc.at[dbufs[nb]], ssem[nb]).wait()

                @pl.when(i + 1 < nwh)
                def _():
                    _fill_win(sbufs[nb], sall, i + 1)
                    pltpu.async_copy(g_hbm.at[sbufs[nb]], rows[nb], gsem[nb])
                    _fill_win(dbufs[nb], dall, i + 1)

                pltpu.make_async_copy(g_hbm.at[sbufs[b]], rows[b],
                                      gsem[b]).wait()
                pltpu.async_copy(rows[b], acc.at[dbufs[b]], ssem[b], add=True)
            return carry

        for h in range(2):
            off = pl.multiple_of(wid * chunk + h * chunk2, WIN)
            pltpu.sync_copy(src_hbm.at[pl.ds(off, chunk2)], sall)
            pltpu.sync_copy(dst_hbm.at[pl.ds(off, chunk2)], dall)
            _fill_win(sb0, sall, 0)
            _fill_win(db0, dall, 0)
            pltpu.async_copy(g_hbm.at[sb0], rows0, gs0)
            lax.fori_loop(0, nwh // 2, body, 0)
            pltpu.make_async_copy(rows1, acc.at[db1], ssem[1]).wait()

        plsc.subcore_barrier()

        for off, sz in _chunks(zrows):
            r0 = s * zrows + off
            pltpu.sync_copy(acc.at[pl.ds(r0, sz)], rows0.at[pl.ds(0, sz)])

            @pl.when(c == 0)
            def _():
                pltpu.sync_copy(rows0.at[pl.ds(0, sz)],
                                out0.at[pl.ds(r0, sz)])

            @pl.when(c == 1)
            def _():
                pltpu.sync_copy(rows0.at[pl.ds(0, sz)],
                                out1.at[pl.ds(r0, sz)])

    return prop_kernel


_BM = 1000


def _tc_call(body, n, out_shapes, in_specs, out_specs):
    return pl.pallas_call(
        body,
        grid=(n // _BM,),
        in_specs=in_specs,
        out_specs=out_specs,
        out_shape=out_shapes,
    )


def _rowspec(c):
    return pl.BlockSpec((_BM, c), lambda i: (i, 0))


def _fullspec(r, c):
    return pl.BlockSpec((r, c), lambda i: (0, 0))


def _dis_body(d0, d1, x, dis_o, g_o):
    deg = d0[:, 0:1] + d1[:, 0:1]
    dis = lax.rsqrt(deg)
    dis_o[...] = dis
    g_o[...] = dis * x[...]


def _mix_chain_body(a0, a1, dis, g_o):
    d = dis[...]
    g_o[...] = d * d * (a0[...] + a1[...])


def _mix_mid_body(a0, a1, dis, h, g_o):
    d = dis[...]
    g_o[...] = d * (0.8 * (d * (a0[...] + a1[...])) + 0.2 * h[...])


def _mix_end_body(a0, a1, dis, h, h_o, g_o):
    d = dis[...]
    z = 0.8 * (d * (a0[...] + a1[...])) + 0.2 * h[...]
    z = jnp.where(z >= 0, z, 0.01 * z)
    h_o[...] = z
    g_o[...] = d * z


def _mm0_body(a0, a1, dis, w, b, h_o, g_o):
    d = dis[...]
    sarr = d * (a0[...] + a1[...])
    h = jnp.dot(sarr, w[...], preferred_element_type=jnp.float32) + b[...]
    h_o[...] = h
    g_o[...] = d * h


def _mmf_body(a0, a1, dis, w, b, o_ref):
    d = dis[...]
    sarr = d * (a0[...] + a1[...])
    h = jnp.dot(sarr, w[...], preferred_element_type=jnp.float32) + b[...]
    m = jnp.max(h, axis=1, keepdims=True)
    z = h - m
    o_ref[...] = z - jnp.log(jnp.sum(jnp.exp(z), axis=1, keepdims=True))


def kernel(x, edge_index, W0, b0, W4, b4):
    n, cdim = x.shape
    e = edge_index.shape[1]
    outc = W4.shape[1]
    f32 = jnp.float32

    ea = e + n
    nw = -(-ea // (NWK * WIN))
    nw = -(-nw // 4) * 4
    ea_pad = NWK * nw * WIN
    npad = ea_pad - ea

    ar = jnp.arange(n, dtype=jnp.int32)
    padi = jnp.arange(npad, dtype=jnp.int32)
    src = jnp.concatenate([edge_index[0], ar, (padi * 7919) % n])
    dst = jnp.concatenate([edge_index[1], ar, n + (padi % GR)])
    z128 = jnp.zeros((WIN, cdim), f32)
    ones16 = jnp.ones((WIN, 16), f32)
    z16 = jnp.zeros((WIN, 16), f32)

    deg0, deg1 = _make_deg(n, nw)(dst, ones16, z16)

    dis, g = _tc_call(
        _dis_body, n,
        [jax.ShapeDtypeStruct((n, 1), f32),
         jax.ShapeDtypeStruct((n, cdim), f32)],
        [_rowspec(16), _rowspec(16), _rowspec(cdim)],
        [_rowspec(1), _rowspec(cdim)],
    )(deg0, deg1, x)

    prop = _make_prop(n, cdim, nw)
    gshape = jax.ShapeDtypeStruct((n, cdim), f32)

    mix_chain = _tc_call(
        _mix_chain_body, n, gshape,
        [_rowspec(cdim), _rowspec(cdim), _rowspec(1)], _rowspec(cdim))
    mix_mid = _tc_call(
        _mix_mid_body, n, gshape,
        [_rowspec(cdim), _rowspec(cdim), _rowspec(1), _rowspec(cdim)],
        _rowspec(cdim))
    mix_end = _tc_call(
        _mix_end_body, n, [gshape, gshape],
        [_rowspec(cdim), _rowspec(cdim), _rowspec(1), _rowspec(cdim)],
        [_rowspec(cdim), _rowspec(cdim)])

    a0, a1 = prop(src, dst, g, z128)
    g = mix_chain(a0, a1, dis)
    a0, a1 = prop(src, dst, g, z128)
    h, g = _tc_call(
        _mm0_body, n, [gshape, gshape],
        [_rowspec(cdim), _rowspec(cdim), _rowspec(1),
         _fullspec(cdim, cdim), _fullspec(1, cdim)],
        [_rowspec(cdim), _rowspec(cdim)],
    )(a0, a1, dis, W0, b0.reshape(1, cdim))

    for _ in range(3):
        a0, a1 = prop(src, dst, g, z128)
        g = mix_mid(a0, a1, dis, h)
        a0, a1 = prop(src, dst, g, z128)
        h, g = mix_end(a0, a1, dis, h)

    a0, a1 = prop(src, dst, g, z128)
    g = mix_chain(a0, a1, dis)
    a0, a1 = prop(src, dst, g, z128)
    out = _tc_call(
        _mmf_body, n, jax.ShapeDtypeStruct((n, outc), f32),
        [_rowspec(cdim), _rowspec(cdim), _rowspec(1),
         _fullspec(cdim, outc), _fullspec(1, outc)],
        _rowspec(outc),
    )(a0, a1, dis, W4, b4.reshape(1, outc))
    return out


# speedup vs baseline: 22.3329x; 1.1217x over previous
import functools

import jax
import jax.numpy as jnp
from jax import lax
from jax.experimental import pallas as pl
from jax.experimental.pallas import tpu as pltpu
from jax.experimental.pallas import tpu_sc as plsc

NC = 2
NS = 16
NWK = NC * NS
WIN = 96
GR = 112


def _chunks(rows):
    out, off = [], 0
    while off < rows:
        sz = min(WIN, rows - off)
        out.append((off, sz))
        off += sz
    return out


def _mesh():
    return plsc.VectorSubcoreMesh(
        core_axis_name="c", subcore_axis_name="s",
        num_cores=NC, num_subcores=NS)


def _fill_win(dst_ref, src_ref, i):
    for q in range(WIN // 16):
        dst_ref[pl.ds(q * 16, 16)] = src_ref[pl.ds(i * WIN + q * 16, 16)]


def _make_deg(n, nw):
    acc_rows = n + GR
    zrows = acc_rows // NS
    chunk = nw * WIN

    @functools.partial(
        pl.kernel, mesh=_mesh(),
        out_type=[jax.ShapeDtypeStruct((acc_rows, 16), jnp.float32),
                  jax.ShapeDtypeStruct((acc_rows, 16), jnp.float32)],
        scratch_types=[
            pltpu.VMEM((WIN, 16), jnp.float32),
            pltpu.VMEM((WIN, 16), jnp.float32),
            pltpu.VMEM((chunk,), jnp.int32),
            pltpu.VMEM((WIN,), jnp.int32),
            pltpu.VMEM((WIN,), jnp.int32),
            pltpu.VMEM_SHARED((acc_rows, 16), jnp.float32),
            pltpu.SemaphoreType.DMA,
            pltpu.SemaphoreType.DMA,
        ],
    )
    def deg_kernel(dst_hbm, ones_hbm, z_hbm, out0, out1,
                   onesb, zb, dall, db0, db1, acc, s0, s1):
        c = lax.axis_index("c")
        s = lax.axis_index("s")
        wid = s * NC + c
        pltpu.sync_copy(ones_hbm, onesb)
        pltpu.sync_copy(z_hbm, zb)
        pltpu.sync_copy(
            dst_hbm.at[pl.ds(pl.multiple_of(wid * chunk, WIN), chunk)], dall)
        for off, sz in _chunks(zrows):
            pltpu.sync_copy(zb.at[pl.ds(0, sz)],
                            acc.at[pl.ds(s * zrows + off, sz)])
        plsc.subcore_barrier()

        dbufs = (db0, db1)
        sems = (s0, s1)

        def body(j, carry):
            for b in range(2):
                i = 2 * j + b

                @pl.when(i >= 2)
                def _():
                    pltpu.make_async_copy(
                        onesb, acc.at[dbufs[b]], sems[b]).wait()

                _fill_win(dbufs[b], dall, i)
                pltpu.async_copy(onesb, acc.at[dbufs[b]], sems[b], add=True)
            return carry

        lax.fori_loop(0, nw // 2, body, 0)
        pltpu.make_async_copy(onesb, acc.at[db0], s0).wait()
        pltpu.make_async_copy(onesb, acc.at[db1], s1).wait()
        plsc.subcore_barrier()

        for off, sz in _chunks(zrows):
            r0 = s * zrows + off
            pltpu.sync_copy(acc.at[pl.ds(r0, sz)], zb.at[pl.ds(0, sz)])

            @pl.when(c == 0)
            def _():
                pltpu.sync_copy(zb.at[pl.ds(0, sz)], out0.at[pl.ds(r0, sz)])

            @pl.when(c == 1)
            def _():
                pltpu.sync_copy(zb.at[pl.ds(0, sz)], out1.at[pl.ds(r0, sz)])

    return deg_kernel


def _make_prop(n, cdim, nw):
    acc_rows = n + GR
    zrows = acc_rows // NS

    chunk = nw * WIN
    nwh = nw // 2
    chunk2 = nwh * WIN

    @functools.partial(
        pl.kernel, mesh=_mesh(),
        out_type=[jax.ShapeDtypeStruct((acc_rows, cdim), jnp.float32),
                  jax.ShapeDtypeStruct((acc_rows, cdim), jnp.float32)],
        scratch_types=[
            pltpu.VMEM((chunk2,), jnp.int32),
            pltpu.VMEM((chunk2,), jnp.int32),
            pltpu.VMEM((WIN,), jnp.int32),
            pltpu.VMEM((WIN,), jnp.int32),
            pltpu.VMEM((WIN,), jnp.int32),
            pltpu.VMEM((WIN,), jnp.int32),
            pltpu.VMEM((WIN,), jnp.int32),
            pltpu.VMEM((WIN,), jnp.int32),
            pltpu.VMEM((WIN, cdim), jnp.float32),
            pltpu.VMEM((WIN, cdim), jnp.float32),
            pltpu.VMEM((WIN, cdim), jnp.float32),
            pltpu.VMEM_SHARED((acc_rows, cdim), jnp.float32),
            pltpu.SemaphoreType.DMA,
            pltpu.SemaphoreType.DMA,
            pltpu.SemaphoreType.DMA,
            pltpu.SemaphoreType.DMA,
            pltpu.SemaphoreType.DMA,
            pltpu.SemaphoreType.DMA,
        ],
    )
    def prop_kernel(src_hbm, dst_hbm, g_hbm, z_hbm, out0, out1,
                    sall, dall, sb0, sb1, sb2, db0, db1, db2,
                    rows0, rows1, rows2, acc,
                    gs0, gs1, gs2, ss0, ss1, ss2):
        c = lax.axis_index("c")
        s = lax.axis_index("s")
        wid = s * NC + c
        pltpu.sync_copy(z_hbm, rows0)
        for off_z, sz in _chunks(zrows):
            pltpu.sync_copy(rows0.at[pl.ds(0, sz)],
                            acc.at[pl.ds(s * zrows + off_z, sz)])
        plsc.subcore_barrier()

        sbufs = (sb0, sb1, sb2)
        dbufs = (db0, db1, db2)
        rows = (rows0, rows1, rows2)
        gsem = (gs0, gs1, gs2)
        ssem = (ss0, ss1, ss2)

        def body(j, carry):
            for t in range(3):
                i = 3 * j + t
                b = t
                b2 = (t + 2) % 3

                @pl.when(i + 2 < nwh)
                def _():
                    @pl.when(i >= 1)
                    def _():
                        pltpu.make_async_copy(
                            rows[b2], acc.at[dbufs[b2]], ssem[b2]).wait()

                    _fill_win(sbufs[b2], sall, i + 2)
                    pltpu.async_copy(g_hbm.at[sbufs[b2]], rows[b2], gsem[b2])
                    _fill_win(dbufs[b2], dall, i + 2)

                pltpu.make_async_copy(g_hbm.at[sbufs[b]], rows[b],
                                      gsem[b]).wait()
                pltpu.async_copy(rows[b], acc.at[dbufs[b]], ssem[b], add=True)
            return carry

        for h in range(2):
            off = pl.multiple_of(wid * chunk + h * chunk2, WIN)
            pltpu.sync_copy(src_hbm.at[pl.ds(off, chunk2)], sall)
            pltpu.sync_copy(dst_hbm.at[pl.ds(off, chunk2)], dall)
            _fill_win(sb0, sall, 0)
            _fill_win(db0, dall, 0)
            pltpu.async_copy(g_hbm.at[sb0], rows0, gs0)
            _fill_win(sb1, sall, 1)
            _fill_win(db1, dall, 1)
            pltpu.async_copy(g_hbm.at[sb1], rows1, gs1)
            lax.fori_loop(0, nwh // 3, body, 0)
            for t in range(3):
                pltpu.make_async_copy(rows[t], acc.at[dbufs[t]],
                                      ssem[t]).wait()

        plsc.subcore_barrier()

        for off, sz in _chunks(zrows):
            r0 = s * zrows + off
            pltpu.sync_copy(acc.at[pl.ds(r0, sz)], rows0.at[pl.ds(0, sz)])

            @pl.when(c == 0)
            def _():
                pltpu.sync_copy(rows0.at[pl.ds(0, sz)],
                                out0.at[pl.ds(r0, sz)])

            @pl.when(c == 1)
            def _():
                pltpu.sync_copy(rows0.at[pl.ds(0, sz)],
                                out1.at[pl.ds(r0, sz)])

    return prop_kernel


_BM = 1000


def _tc_call(body, n, out_shapes, in_specs, out_specs):
    return pl.pallas_call(
        body,
        grid=(n // _BM,),
        in_specs=in_specs,
        out_specs=out_specs,
        out_shape=out_shapes,
    )


def _rowspec(c):
    return pl.BlockSpec((_BM, c), lambda i: (i, 0))


def _fullspec(r, c):
    return pl.BlockSpec((r, c), lambda i: (0, 0))


def _dis_body(d0, d1, x, dis_o, g_o):
    deg = d0[:, 0:1] + d1[:, 0:1]
    dis = lax.rsqrt(deg)
    dis_o[...] = dis
    g_o[...] = dis * x[...]


def _mix_chain_body(a0, a1, dis, g_o):
    d = dis[...]
    g_o[...] = d * d * (a0[...] + a1[...])


def _mix_mid_body(a0, a1, dis, h, g_o):
    d = dis[...]
    g_o[...] = d * (0.8 * (d * (a0[...] + a1[...])) + 0.2 * h[...])


def _mix_end_body(a0, a1, dis, h, h_o, g_o):
    d = dis[...]
    z = 0.8 * (d * (a0[...] + a1[...])) + 0.2 * h[...]
    z = jnp.where(z >= 0, z, 0.01 * z)
    h_o[...] = z
    g_o[...] = d * z


def _mm0_body(a0, a1, dis, w, b, h_o, g_o):
    d = dis[...]
    sarr = d * (a0[...] + a1[...])
    h = jnp.dot(sarr, w[...], preferred_element_type=jnp.float32) + b[...]
    h_o[...] = h
    g_o[...] = d * h


def _mmf_body(a0, a1, dis, w, b, o_ref):
    d = dis[...]
    sarr = d * (a0[...] + a1[...])
    h = jnp.dot(sarr, w[...], preferred_element_type=jnp.float32) + b[...]
    m = jnp.max(h, axis=1, keepdims=True)
    z = h - m
    o_ref[...] = z - jnp.log(jnp.sum(jnp.exp(z), axis=1, keepdims=True))


def kernel(x, edge_index, W0, b0, W4, b4):
    n, cdim = x.shape
    e = edge_index.shape[1]
    outc = W4.shape[1]
    f32 = jnp.float32

    ea = e + n
    nw = -(-ea // (NWK * WIN))
    nw = -(-nw // 6) * 6
    ea_pad = NWK * nw * WIN
    npad = ea_pad - ea

    ar = jnp.arange(n, dtype=jnp.int32)
    padi = jnp.arange(npad, dtype=jnp.int32)
    src = jnp.concatenate([edge_index[0], ar, (padi * 7919) % n])
    dst = jnp.concatenate([edge_index[1], ar, n + (padi % GR)])
    z128 = jnp.zeros((WIN, cdim), f32)
    ones16 = jnp.ones((WIN, 16), f32)
    z16 = jnp.zeros((WIN, 16), f32)

    deg0, deg1 = _make_deg(n, nw)(dst, ones16, z16)

    dis, g = _tc_call(
        _dis_body, n,
        [jax.ShapeDtypeStruct((n, 1), f32),
         jax.ShapeDtypeStruct((n, cdim), f32)],
        [_rowspec(16), _rowspec(16), _rowspec(cdim)],
        [_rowspec(1), _rowspec(cdim)],
    )(deg0, deg1, x)

    prop = _make_prop(n, cdim, nw)
    gshape = jax.ShapeDtypeStruct((n, cdim), f32)

    mix_chain = _tc_call(
        _mix_chain_body, n, gshape,
        [_rowspec(cdim), _rowspec(cdim), _rowspec(1)], _rowspec(cdim))
    mix_mid = _tc_call(
        _mix_mid_body, n, gshape,
        [_rowspec(cdim), _rowspec(cdim), _rowspec(1), _rowspec(cdim)],
        _rowspec(cdim))
    mix_end = _tc_call(
        _mix_end_body, n, [gshape, gshape],
        [_rowspec(cdim), _rowspec(cdim), _rowspec(1), _rowspec(cdim)],
        [_rowspec(cdim), _rowspec(cdim)])

    a0, a1 = prop(src, dst, g, z128)
    g = mix_chain(a0, a1, dis)
    a0, a1 = prop(src, dst, g, z128)
    h, g = _tc_call(
        _mm0_body, n, [gshape, gshape],
        [_rowspec(cdim), _rowspec(cdim), _rowspec(1),
         _fullspec(cdim, cdim), _fullspec(1, cdim)],
        [_rowspec(cdim), _rowspec(cdim)],
    )(a0, a1, dis, W0, b0.reshape(1, cdim))

    for _ in range(3):
        a0, a1 = prop(src, dst, g, z128)
        g = mix_mid(a0, a1, dis, h)
        a0, a1 = prop(src, dst, g, z128)
        h, g = mix_end(a0, a1, dis, h)

    a0, a1 = prop(src, dst, g, z128)
    g = mix_chain(a0, a1, dis)
    a0, a1 = prop(src, dst, g, z128)
    out = _tc_call(
        _mmf_body, n, jax.ShapeDtypeStruct((n, outc), f32),
        [_rowspec(cdim), _rowspec(cdim), _rowspec(1),
         _fullspec(cdim, outc), _fullspec(1, outc)],
        _rowspec(outc),
    )(a0, a1, dis, W4, b4.reshape(1, outc))
    return out


# speedup vs baseline: 23.1923x; 1.0385x over previous
import functools

import jax
import jax.numpy as jnp
from jax import lax
from jax.experimental import pallas as pl
from jax.experimental.pallas import tpu as pltpu
from jax.experimental.pallas import tpu_sc as plsc

NC = 2
NS = 16
NWK = NC * NS
WIN = 96
GR = 112


def _chunks(rows):
    out, off = [], 0
    while off < rows:
        sz = min(WIN, rows - off)
        out.append((off, sz))
        off += sz
    return out


def _mesh():
    return plsc.VectorSubcoreMesh(
        core_axis_name="c", subcore_axis_name="s",
        num_cores=NC, num_subcores=NS)


def _fill_win(dst_ref, src_ref, i):
    for q in range(WIN // 16):
        dst_ref[pl.ds(q * 16, 16)] = src_ref[pl.ds(i * WIN + q * 16, 16)]


def _make_deg(n, nw):
    acc_rows = n + GR
    zrows = acc_rows // NS
    chunk = nw * WIN

    @functools.partial(
        pl.kernel, mesh=_mesh(),
        out_type=[jax.ShapeDtypeStruct((acc_rows, 16), jnp.float32),
                  jax.ShapeDtypeStruct((acc_rows, 16), jnp.float32)],
        scratch_types=[
            pltpu.VMEM((WIN, 16), jnp.float32),
            pltpu.VMEM((WIN, 16), jnp.float32),
            pltpu.VMEM((chunk,), jnp.int32),
            pltpu.VMEM((WIN,), jnp.int32),
            pltpu.VMEM((WIN,), jnp.int32),
            pltpu.VMEM_SHARED((acc_rows, 16), jnp.float32),
            pltpu.SemaphoreType.DMA,
            pltpu.SemaphoreType.DMA,
        ],
    )
    def deg_kernel(dst_hbm, ones_hbm, z_hbm, out0, out1,
                   onesb, zb, dall, db0, db1, acc, s0, s1):
        c = lax.axis_index("c")
        s = lax.axis_index("s")
        wid = s * NC + c
        pltpu.sync_copy(ones_hbm, onesb)
        pltpu.sync_copy(z_hbm, zb)
        pltpu.sync_copy(
            dst_hbm.at[pl.ds(pl.multiple_of(wid * chunk, WIN), chunk)], dall)
        for off, sz in _chunks(zrows):
            pltpu.sync_copy(zb.at[pl.ds(0, sz)],
                            acc.at[pl.ds(s * zrows + off, sz)])
        plsc.subcore_barrier()

        dbufs = (db0, db1)
        sems = (s0, s1)

        def body(j, carry):
            for b in range(2):
                i = 2 * j + b

                @pl.when(i >= 2)
                def _():
                    pltpu.make_async_copy(
                        onesb, acc.at[dbufs[b]], sems[b]).wait()

                _fill_win(dbufs[b], dall, i)
                pltpu.async_copy(onesb, acc.at[dbufs[b]], sems[b], add=True)
            return carry

        lax.fori_loop(0, nw // 2, body, 0)
        pltpu.make_async_copy(onesb, acc.at[db0], s0).wait()
        pltpu.make_async_copy(onesb, acc.at[db1], s1).wait()
        plsc.subcore_barrier()

        for off, sz in _chunks(zrows):
            r0 = s * zrows + off
            pltpu.sync_copy(acc.at[pl.ds(r0, sz)], zb.at[pl.ds(0, sz)])

            @pl.when(c == 0)
            def _():
                pltpu.sync_copy(zb.at[pl.ds(0, sz)], out0.at[pl.ds(r0, sz)])

            @pl.when(c == 1)
            def _():
                pltpu.sync_copy(zb.at[pl.ds(0, sz)], out1.at[pl.ds(r0, sz)])

    return deg_kernel


def _make_prop(n, cdim, nw, tc_tiling=True):
    acc_rows = n + GR
    zrows = acc_rows // NS

    chunk = nw * WIN
    nwh = nw // 2
    chunk2 = nwh * WIN

    @functools.partial(
        pl.kernel, mesh=_mesh(),
        out_type=[jax.ShapeDtypeStruct((acc_rows, cdim), jnp.float32),
                  jax.ShapeDtypeStruct((acc_rows, cdim), jnp.float32)],
        compiler_params=pltpu.CompilerParams(use_tc_tiling_on_sc=tc_tiling),
        scratch_types=[
            pltpu.VMEM((chunk2,), jnp.int32),
            pltpu.VMEM((chunk2,), jnp.int32),
            pltpu.VMEM((WIN,), jnp.int32),
            pltpu.VMEM((WIN,), jnp.int32),
            pltpu.VMEM((WIN,), jnp.int32),
            pltpu.VMEM((WIN,), jnp.int32),
            pltpu.VMEM((WIN,), jnp.int32),
            pltpu.VMEM((WIN,), jnp.int32),
            pltpu.VMEM((WIN, cdim), jnp.float32),
            pltpu.VMEM((WIN, cdim), jnp.float32),
            pltpu.VMEM((WIN, cdim), jnp.float32),
            pltpu.VMEM_SHARED((acc_rows, cdim), jnp.float32),
            pltpu.SemaphoreType.DMA,
            pltpu.SemaphoreType.DMA,
            pltpu.SemaphoreType.DMA,
            pltpu.SemaphoreType.DMA,
            pltpu.SemaphoreType.DMA,
            pltpu.SemaphoreType.DMA,
        ],
    )
    def prop_kernel(src_hbm, dst_hbm, g_hbm, z_hbm, out0, out1,
                    sall, dall, sb0, sb1, sb2, db0, db1, db2,
                    rows0, rows1, rows2, acc,
                    gs0, gs1, gs2, ss0, ss1, ss2):
        c = lax.axis_index("c")
        s = lax.axis_index("s")
        wid = s * NC + c
        pltpu.sync_copy(z_hbm, rows0)
        for off_z, sz in _chunks(zrows):
            pltpu.sync_copy(rows0.at[pl.ds(0, sz)],
                            acc.at[pl.ds(s * zrows + off_z, sz)])
        plsc.subcore_barrier()

        sbufs = (sb0, sb1, sb2)
        dbufs = (db0, db1, db2)
        rows = (rows0, rows1, rows2)
        gsem = (gs0, gs1, gs2)
        ssem = (ss0, ss1, ss2)

        def body(j, carry):
            for t in range(3):
                i = 3 * j + t
                b = t
                b2 = (t + 2) % 3

                @pl.when(i + 2 < nwh)
                def _():
                    @pl.when(i >= 1)
                    def _():
                        pltpu.make_async_copy(
                            rows[b2], acc.at[dbufs[b2]], ssem[b2]).wait()

                    _fill_win(sbufs[b2], sall, i + 2)
                    pltpu.async_copy(g_hbm.at[sbufs[b2]], rows[b2], gsem[b2])
                    _fill_win(dbufs[b2], dall, i + 2)

                pltpu.make_async_copy(g_hbm.at[sbufs[b]], rows[b],
                                      gsem[b]).wait()
                pltpu.async_copy(rows[b], acc.at[dbufs[b]], ssem[b], add=True)
            return carry

        for h in range(2):
            off = pl.multiple_of(wid * chunk + h * chunk2, WIN)
            pltpu.sync_copy(src_hbm.at[pl.ds(off, chunk2)], sall)
            pltpu.sync_copy(dst_hbm.at[pl.ds(off, chunk2)], dall)
            _fill_win(sb0, sall, 0)
            _fill_win(db0, dall, 0)
            pltpu.async_copy(g_hbm.at[sb0], rows0, gs0)
            _fill_win(sb1, sall, 1)
            _fill_win(db1, dall, 1)
            pltpu.async_copy(g_hbm.at[sb1], rows1, gs1)
            lax.fori_loop(0, nwh // 3, body, 0)
            for t in range(3):
                pltpu.make_async_copy(rows[t], acc.at[dbufs[t]],
                                      ssem[t]).wait()

        plsc.subcore_barrier()

        for off, sz in _chunks(zrows):
            r0 = s * zrows + off
            pltpu.sync_copy(acc.at[pl.ds(r0, sz)], rows0.at[pl.ds(0, sz)])

            @pl.when(c == 0)
            def _():
                pltpu.sync_copy(rows0.at[pl.ds(0, sz)],
                                out0.at[pl.ds(r0, sz)])

            @pl.when(c == 1)
            def _():
                pltpu.sync_copy(rows0.at[pl.ds(0, sz)],
                                out1.at[pl.ds(r0, sz)])

    return prop_kernel


_BM = 1000


def _tc_call(body, n, out_shapes, in_specs, out_specs):
    return pl.pallas_call(
        body,
        grid=(n // _BM,),
        in_specs=in_specs,
        out_specs=out_specs,
        out_shape=out_shapes,
    )


def _rowspec(c):
    return pl.BlockSpec((_BM, c), lambda i: (i, 0))


def _fullspec(r, c):
    return pl.BlockSpec((r, c), lambda i: (0, 0))


def _dis_body(d0, d1, x, dis_o, g_o):
    deg = d0[:, 0:1] + d1[:, 0:1]
    dis = lax.rsqrt(deg)
    dis_o[...] = dis
    g_o[...] = dis * x[...]


def _mix_chain_body(a0, a1, dis, g_o):
    d = dis[...]
    g_o[...] = d * d * (a0[...] + a1[...])


def _mix_mid_body(a0, a1, dis, h, g_o):
    d = dis[...]
    g_o[...] = d * (0.8 * (d * (a0[...] + a1[...])) + 0.2 * h[...])


def _mix_end_body(a0, a1, dis, h, h_o, g_o):
    d = dis[...]
    z = 0.8 * (d * (a0[...] + a1[...])) + 0.2 * h[...]
    z = jnp.where(z >= 0, z, 0.01 * z)
    h_o[...] = z
    g_o[...] = d * z


def _mm4_body(a0, a1, dis, h, w, g_o):
    d = dis[...]
    z = 0.8 * (d * (a0[...] + a1[...])) + 0.2 * h[...]
    z = jnp.where(z >= 0, z, 0.01 * z)
    g_o[...] = d * jnp.dot(z, w[...], preferred_element_type=jnp.float32)


def _mm0_body(a0, a1, dis, w, b, h_o, g_o):
    d = dis[...]
    sarr = d * (a0[...] + a1[...])
    h = jnp.dot(sarr, w[...], preferred_element_type=jnp.float32) + b[...]
    h_o[...] = h
    g_o[...] = d * h


def _mmf_body(a0, a1, dis, b, o_ref):
    d = dis[...]
    h = d * (a0[...] + a1[...]) + b[...]
    m = jnp.max(h, axis=1, keepdims=True)
    z = h - m
    o_ref[...] = z - jnp.log(jnp.sum(jnp.exp(z), axis=1, keepdims=True))


def kernel(x, edge_index, W0, b0, W4, b4):
    n, cdim = x.shape
    e = edge_index.shape[1]
    outc = W4.shape[1]
    f32 = jnp.float32

    ea = e + n
    nw = -(-ea // (NWK * WIN))
    nw = -(-nw // 6) * 6
    ea_pad = NWK * nw * WIN
    npad = ea_pad - ea

    ar = jnp.arange(n, dtype=jnp.int32)
    padi = jnp.arange(npad, dtype=jnp.int32)
    src = jnp.concatenate([edge_index[0], ar, (padi * 7919) % n])
    dst = jnp.concatenate([edge_index[1], ar, n + (padi % GR)])
    z128 = jnp.zeros((WIN, cdim), f32)
    ones16 = jnp.ones((WIN, 16), f32)
    z16 = jnp.zeros((WIN, 16), f32)

    deg0, deg1 = _make_deg(n, nw)(dst, ones16, z16)

    dis, g = _tc_call(
        _dis_body, n,
        [jax.ShapeDtypeStruct((n, 1), f32),
         jax.ShapeDtypeStruct((n, cdim), f32)],
        [_rowspec(16), _rowspec(16), _rowspec(cdim)],
        [_rowspec(1), _rowspec(cdim)],
    )(deg0, deg1, x)

    prop = _make_prop(n, cdim, nw)
    gshape = jax.ShapeDtypeStruct((n, cdim), f32)

    mix_chain = _tc_call(
        _mix_chain_body, n, gshape,
        [_rowspec(cdim), _rowspec(cdim), _rowspec(1)], _rowspec(cdim))
    mix_mid = _tc_call(
        _mix_mid_body, n, gshape,
        [_rowspec(cdim), _rowspec(cdim), _rowspec(1), _rowspec(cdim)],
        _rowspec(cdim))
    mix_end = _tc_call(
        _mix_end_body, n, [gshape, gshape],
        [_rowspec(cdim), _rowspec(cdim), _rowspec(1), _rowspec(cdim)],
        [_rowspec(cdim), _rowspec(cdim)])

    a0, a1 = prop(src, dst, g, z128)
    g = mix_chain(a0, a1, dis)
    a0, a1 = prop(src, dst, g, z128)
    h, g = _tc_call(
        _mm0_body, n, [gshape, gshape],
        [_rowspec(cdim), _rowspec(cdim), _rowspec(1),
         _fullspec(cdim, cdim), _fullspec(1, cdim)],
        [_rowspec(cdim), _rowspec(cdim)],
    )(a0, a1, dis, W0, b0.reshape(1, cdim))

    for blk in range(3):
        a0, a1 = prop(src, dst, g, z128)
        g = mix_mid(a0, a1, dis, h)
        a0, a1 = prop(src, dst, g, z128)
        if blk < 2:
            h, g = mix_end(a0, a1, dis, h)
        else:
            g = _tc_call(
                _mm4_body, n, jax.ShapeDtypeStruct((n, outc), f32),
                [_rowspec(cdim), _rowspec(cdim), _rowspec(1), _rowspec(cdim),
                 _fullspec(cdim, outc)],
                _rowspec(outc),
            )(a0, a1, dis, h, W4)

    prop64 = _make_prop(n, outc, nw, tc_tiling=False)
    z64 = jnp.zeros((WIN, outc), f32)
    a0, a1 = prop64(src, dst, g, z64)
    g = _tc_call(
        _mix_chain_body, n, jax.ShapeDtypeStruct((n, outc), f32),
        [_rowspec(outc), _rowspec(outc), _rowspec(1)],
        _rowspec(outc))(a0, a1, dis)
    a0, a1 = prop64(src, dst, g, z64)
    out = _tc_call(
        _mmf_body, n, jax.ShapeDtypeStruct((n, outc), f32),
        [_rowspec(outc), _rowspec(outc), _rowspec(1), _fullspec(1, outc)],
        _rowspec(outc),
    )(a0, a1, dis, b4.reshape(1, outc))
    return out
